# initial kernel scaffold (unmeasured)
import jax
import jax.numpy as jnp
from jax import lax
from jax.experimental import pallas as pl
from jax.experimental.pallas import tpu as pltpu


def kernel(
    x,
):
    def body(*refs):
        pass

    out_shape = jax.ShapeDtypeStruct(..., jnp.float32)
    return pl.pallas_call(body, out_shape=out_shape)(...)


# baseline (device time: 306407 ns/iter reference)
import jax
import jax.numpy as jnp
from jax import lax
from jax.experimental import pallas as pl
from jax.experimental.pallas import tpu as pltpu

N_Z = 4


def kernel(x):
    m, n = x.shape
    chunk = m // N_Z

    def body(x_ref, out_ref, comm_ref, rs_send_sems, rs_recv_sems,
             ag_send_sems, ag_recv_sems):
        my_x = lax.axis_index("x")
        my_y = lax.axis_index("y")
        my_z = lax.axis_index("z")
        nxt = (my_z + 1) % N_Z

        out_ref[...] = x_ref[...]

        for s in range(N_Z - 1):
            send_idx = (my_z - s) % N_Z
            recv_idx = (my_z - s - 1) % N_Z
            rdma = pltpu.make_async_remote_copy(
                src_ref=out_ref.at[pl.ds(send_idx * chunk, chunk), :],
                dst_ref=comm_ref.at[s],
                send_sem=rs_send_sems.at[s],
                recv_sem=rs_recv_sems.at[s],
                device_id=(my_x, my_y, nxt),
                device_id_type=pl.DeviceIdType.MESH,
            )
            rdma.start()
            rdma.wait()
            out_ref[pl.ds(recv_idx * chunk, chunk), :] += comm_ref[s]

        for s in range(N_Z - 1):
            send_idx = (my_z + 1 - s) % N_Z
            rdma = pltpu.make_async_remote_copy(
                src_ref=out_ref.at[pl.ds(send_idx * chunk, chunk), :],
                dst_ref=out_ref.at[pl.ds(send_idx * chunk, chunk), :],
                send_sem=ag_send_sems.at[s],
                recv_sem=ag_recv_sems.at[s],
                device_id=(my_x, my_y, nxt),
                device_id_type=pl.DeviceIdType.MESH,
            )
            rdma.start()
            rdma.wait()

    return pl.pallas_call(
        body,
        out_shape=jax.ShapeDtypeStruct((m, n), x.dtype),
        in_specs=[pl.BlockSpec(memory_space=pltpu.VMEM)],
        out_specs=pl.BlockSpec(memory_space=pltpu.VMEM),
        scratch_shapes=[
            pltpu.VMEM((N_Z - 1, chunk, n), x.dtype),
            pltpu.SemaphoreType.DMA((N_Z - 1,)),
            pltpu.SemaphoreType.DMA((N_Z - 1,)),
            pltpu.SemaphoreType.DMA((N_Z - 1,)),
            pltpu.SemaphoreType.DMA((N_Z - 1,)),
        ],
    )(x)


# device time: 195088 ns/iter; 1.5706x vs baseline; 1.5706x over previous
import jax
import jax.numpy as jnp
from jax import lax
from jax.experimental import pallas as pl
from jax.experimental.pallas import tpu as pltpu

N_Z = 4


def kernel(x):
    m, n = x.shape
    half = m // 2
    chunk = half // N_Z

    def body(x_ref, out_ref, comm_ref, rs_send_sems, rs_recv_sems,
             ag_send_sems, ag_recv_sems, px_send_sems, px_recv_sems):
        my_x = lax.axis_index("x")
        my_y = lax.axis_index("y")
        my_z = lax.axis_index("z")
        nxt = (my_z + 1) % N_Z
        partner = (1 - my_x, my_y, my_z)
        base = my_x * half

        def rows(c):
            return pl.ds(base + c * chunk, chunk)

        out_ref[pl.ds(base, half), :] = x_ref[pl.ds(base, half), :]

        for s in range(N_Z - 1):
            send_c = (my_z - s) % N_Z
            recv_c = (my_z - s - 1) % N_Z
            rdma = pltpu.make_async_remote_copy(
                src_ref=out_ref.at[rows(send_c), :],
                dst_ref=comm_ref.at[s],
                send_sem=rs_send_sems.at[s],
                recv_sem=rs_recv_sems.at[s],
                device_id=(my_x, my_y, nxt),
                device_id_type=pl.DeviceIdType.MESH,
            )
            rdma.start()
            rdma.wait()
            out_ref[rows(recv_c), :] += comm_ref[s]

        x_rdmas = []
        c0 = (my_z + 1) % N_Z
        xr = pltpu.make_async_remote_copy(
            src_ref=out_ref.at[rows(c0), :],
            dst_ref=out_ref.at[rows(c0), :],
            send_sem=px_send_sems.at[0],
            recv_sem=px_recv_sems.at[0],
            device_id=partner,
            device_id_type=pl.DeviceIdType.MESH,
        )
        xr.start()
        x_rdmas.append(xr)

        for s in range(N_Z - 1):
            send_c = (my_z + 1 - s) % N_Z
            recv_c = (my_z - s) % N_Z
            rdma = pltpu.make_async_remote_copy(
                src_ref=out_ref.at[rows(send_c), :],
                dst_ref=out_ref.at[rows(send_c), :],
                send_sem=ag_send_sems.at[s],
                recv_sem=ag_recv_sems.at[s],
                device_id=(my_x, my_y, nxt),
                device_id_type=pl.DeviceIdType.MESH,
            )
            rdma.start()
            rdma.wait()
            xr = pltpu.make_async_remote_copy(
                src_ref=out_ref.at[rows(recv_c), :],
                dst_ref=out_ref.at[rows(recv_c), :],
                send_sem=px_send_sems.at[s + 1],
                recv_sem=px_recv_sems.at[s + 1],
                device_id=partner,
                device_id_type=pl.DeviceIdType.MESH,
            )
            xr.start()
            x_rdmas.append(xr)

        for xr in x_rdmas:
            xr.wait()

    return pl.pallas_call(
        body,
        out_shape=jax.ShapeDtypeStruct((m, n), x.dtype),
        in_specs=[pl.BlockSpec(memory_space=pltpu.VMEM)],
        out_specs=pl.BlockSpec(memory_space=pltpu.VMEM),
        scratch_shapes=[
            pltpu.VMEM((N_Z - 1, chunk, n), x.dtype),
            pltpu.SemaphoreType.DMA((N_Z - 1,)),
            pltpu.SemaphoreType.DMA((N_Z - 1,)),
            pltpu.SemaphoreType.DMA((N_Z - 1,)),
            pltpu.SemaphoreType.DMA((N_Z - 1,)),
            pltpu.SemaphoreType.DMA((N_Z,)),
            pltpu.SemaphoreType.DMA((N_Z,)),
        ],
    )(x)


# device time: 138735 ns/iter; 2.2086x vs baseline; 1.4062x over previous
import jax
import jax.numpy as jnp
from jax import lax
from jax.experimental import pallas as pl
from jax.experimental.pallas import tpu as pltpu

N_Z = 4


def kernel(x):
    m, n = x.shape
    qrows = m // 4
    chunk = qrows // N_Z
    sub = chunk // 2

    def body(x_ref, out_ref, comm_ref,
             rs_send, rs_recv, ag_send, ag_recv,
             xq_send, xq_recv, yq_send, yq_recv,
             xf_send, xf_recv, yf_send, yf_recv):
        my_x = lax.axis_index("x")
        my_y = lax.axis_index("y")
        my_z = lax.axis_index("z")
        nxt = (my_z + 1) % N_Z
        h = my_y % 2
        partner = (1 - my_x, my_y, my_z)
        ypair = (my_x, jnp.bitwise_xor(my_y, 1), my_z)

        q_me = 2 * my_x + h
        q_xp = 2 * (1 - my_x) + h
        q_yh = 2 * my_x + (1 - h)

        def rows(q, c, off=0, size=chunk):
            return pl.ds(q * qrows + c * chunk + off, size)

        own = [(my_z + 1) % N_Z, my_z, (my_z - 1) % N_Z, (my_z - 2) % N_Z]

        out_ref[rows(q_me, 0, 0, qrows), :] = x_ref[rows(q_me, 0, 0, qrows), :]

        for s in range(N_Z - 1):
            send_c = (my_z - s) % N_Z
            recv_c = (my_z - s - 1) % N_Z
            rdma = pltpu.make_async_remote_copy(
                src_ref=out_ref.at[rows(q_me, send_c), :],
                dst_ref=comm_ref.at[s],
                send_sem=rs_send.at[s],
                recv_sem=rs_recv.at[s],
                device_id=(my_x, my_y, nxt),
                device_id_type=pl.DeviceIdType.MESH,
            )
            rdma.start()
            rdma.wait()
            out_ref[rows(q_me, recv_c), :] += comm_ref[s]

        xq = [None] * N_Z
        yq = [None] * N_Z
        xf = [None] * N_Z
        yf = [None] * N_Z

        def publish(j):
            c = own[j]
            xq[j] = pltpu.make_async_remote_copy(
                src_ref=out_ref.at[rows(q_me, c), :],
                dst_ref=out_ref.at[rows(q_me, c), :],
                send_sem=xq_send.at[j], recv_sem=xq_recv.at[j],
                device_id=partner, device_id_type=pl.DeviceIdType.MESH,
            )
            xq[j].start()
            yq[j] = pltpu.make_async_remote_copy(
                src_ref=out_ref.at[rows(q_me, c), :],
                dst_ref=out_ref.at[rows(q_me, c), :],
                send_sem=yq_send.at[j], recv_sem=yq_recv.at[j],
                device_id=ypair, device_id_type=pl.DeviceIdType.MESH,
            )
            yq[j].start()

        def forward(j):
            c = own[j]
            yq[j].wait_recv()
            xf[j] = pltpu.make_async_remote_copy(
                src_ref=out_ref.at[rows(q_yh, c, 0, sub), :],
                dst_ref=out_ref.at[rows(q_yh, c, 0, sub), :],
                send_sem=xf_send.at[j], recv_sem=xf_recv.at[j],
                device_id=partner, device_id_type=pl.DeviceIdType.MESH,
            )
            xf[j].start()
            xq[j].wait_recv()
            yf[j] = pltpu.make_async_remote_copy(
                src_ref=out_ref.at[rows(q_xp, c, sub, sub), :],
                dst_ref=out_ref.at[rows(q_xp, c, sub, sub), :],
                send_sem=yf_send.at[j], recv_sem=yf_recv.at[j],
                device_id=ypair, device_id_type=pl.DeviceIdType.MESH,
            )
            yf[j].start()

        publish(0)

        for s in range(N_Z - 1):
            rdma = pltpu.make_async_remote_copy(
                src_ref=out_ref.at[rows(q_me, own[s]), :],
                dst_ref=out_ref.at[rows(q_me, own[s]), :],
                send_sem=ag_send.at[s], recv_sem=ag_recv.at[s],
                device_id=(my_x, my_y, nxt),
                device_id_type=pl.DeviceIdType.MESH,
            )
            rdma.start()
            forward(s)
            rdma.wait()
            publish(s + 1)

        forward(N_Z - 1)

        for j in range(N_Z):
            xq[j].wait_send()
            yq[j].wait_send()
            xf[j].wait()
            yf[j].wait()

    return pl.pallas_call(
        body,
        out_shape=jax.ShapeDtypeStruct((m, n), x.dtype),
        in_specs=[pl.BlockSpec(memory_space=pltpu.VMEM)],
        out_specs=pl.BlockSpec(memory_space=pltpu.VMEM),
        scratch_shapes=[
            pltpu.VMEM((N_Z - 1, chunk, n), x.dtype),
            pltpu.SemaphoreType.DMA((N_Z - 1,)),
            pltpu.SemaphoreType.DMA((N_Z - 1,)),
            pltpu.SemaphoreType.DMA((N_Z - 1,)),
            pltpu.SemaphoreType.DMA((N_Z - 1,)),
            pltpu.SemaphoreType.DMA((N_Z,)),
            pltpu.SemaphoreType.DMA((N_Z,)),
            pltpu.SemaphoreType.DMA((N_Z,)),
            pltpu.SemaphoreType.DMA((N_Z,)),
            pltpu.SemaphoreType.DMA((N_Z,)),
            pltpu.SemaphoreType.DMA((N_Z,)),
            pltpu.SemaphoreType.DMA((N_Z,)),
            pltpu.SemaphoreType.DMA((N_Z,)),
        ],
    )(x)
